# baseline (speedup 1.0000x reference)
import functools

import jax
import jax.numpy as jnp
from jax import lax
from jax.experimental import pallas as pl
from jax.experimental.pallas import tpu as pltpu
from jax.experimental.pallas import tpu_sc as plsc

N_ROWS = 1024
N_CLASSES = 100000
IGNORE = 0
SMOOTH = 0.1
FILL = SMOOTH / (N_CLASSES - 1)
CONF = 1.0 - SMOOTH
DELTA = CONF - FILL

_NC = 2
_NS = 16
_NW = _NC * _NS
RPW = 8
SC_ROWS = RPW * _NW // 2
TC_ROWS = N_ROWS - SC_ROWS

CW = 2048
NCHUNK = 48
HCHUNK = NCHUNK // 2
SC_COLS = NCHUNK * CW
STRIP = N_CLASSES - SC_COLS
NSLOT = _NW * RPW

RB = 16
GRID = TC_ROWS // RB
NBUF = 4
LANE = 128
WIN = 512
TSTART = 99584
TW = N_CLASSES - TSTART
SMAXD = TSTART - WIN
CPB = 6


@functools.cache
def _get_sc_slab():
    return pl.kernel(
        _sc_slab_body,
        mesh=plsc.VectorSubcoreMesh(core_axis_name="c", subcore_axis_name="s"),
        out_type=jax.ShapeDtypeStruct((NSLOT * 16,), jnp.float32),
        scratch_types=[
            pltpu.VMEM((2, RPW, CW), jnp.float32),
            pltpu.VMEM((16,), jnp.float32),
            pltpu.SemaphoreType.DMA((2,)),
        ],
    )


def _sc_slab_body(logit_hbm, out_hbm, chunk_v, out_v, sems):
    wid = lax.axis_index("s") * _NC + lax.axis_index("c")
    base = TC_ROWS + (wid // 2) * RPW
    cbase = (wid % 2) * (HCHUNK * CW)

    def _start(c):
        pltpu.make_async_copy(
            logit_hbm.at[
                pl.ds(base, RPW),
                pl.ds(pl.multiple_of(cbase + c * CW, LANE), CW),
            ],
            chunk_v.at[c % 2],
            sems.at[c % 2],
        ).start()

    _start(0)
    rowacc = [jnp.zeros((16,), jnp.float32) for _ in range(RPW)]
    for c in range(HCHUNK):
        pltpu.make_async_copy(
            logit_hbm.at[
                pl.ds(base, RPW),
                pl.ds(pl.multiple_of(cbase + c * CW, LANE), CW),
            ],
            chunk_v.at[c % 2],
            sems.at[c % 2],
        ).wait()
        if c + 1 < HCHUNK:
            _start(c + 1)
        for k in range(RPW):
            def body(j, acc):
                b = pl.multiple_of(j * 64, 16)
                v = chunk_v[c % 2, k, pl.ds(b, 16)]
                v = v + chunk_v[c % 2, k, pl.ds(b + 16, 16)]
                v = v + chunk_v[c % 2, k, pl.ds(b + 32, 16)]
                v = v + chunk_v[c % 2, k, pl.ds(b + 48, 16)]
                return acc + v

            rowacc[k] = lax.fori_loop(0, CW // 64, body, rowacc[k])

    for k in range(RPW):
        out_v[...] = rowacc[k]
        pltpu.sync_copy(out_v, out_hbm.at[pl.ds((wid * RPW + k) * 16, 16)])


def _copy_in(logit_hbm, buf, sems, blk, slot):
    pltpu.make_async_copy(
        logit_hbm.at[pl.ds(blk * RB, RB), :],
        buf.at[slot],
        sems.at[slot],
    ).start()


def _corr_dma(logit_hbm, corrbuf, csem, tgt_sref, idx):
    t_r = tgt_sref[TC_ROWS + idx]
    tcol = pl.multiple_of(
        jnp.minimum((t_r // LANE) * LANE, SC_COLS - LANE), LANE
    )
    return pltpu.make_async_copy(
        logit_hbm.at[pl.ds(TC_ROWS + (idx // 8) * 8, 8), pl.ds(tcol, LANE)],
        corrbuf.at[idx],
        csem,
    )


def _loss_body(tgt_sref, logit_hbm, tgt_ref, strip_ref, tsc_ref, out_ref,
               buf, sems, corrbuf, csem):
    i = pl.program_id(0)
    slot = jax.lax.rem(i, NBUF)

    @pl.when(i == 0)
    def _():
        out_ref[0, 0] = 0.0
        for b in range(NBUF):
            _copy_in(logit_hbm, buf, sems, jnp.int32(b), jnp.int32(b))

    pltpu.make_async_copy(
        logit_hbm.at[pl.ds(i * RB, RB), :],
        buf.at[slot],
        sems.at[slot],
    ).wait()

    x = buf[slot]

    rs_row = jnp.sum(x, axis=1, keepdims=True)
    t = tgt_ref[...]
    fill_row = jnp.where(t == IGNORE, 0.0, FILL)
    fill_part = jnp.sum(fill_row * rs_row)

    lane_iota = jax.lax.broadcasted_iota(jnp.int32, (1, WIN), 1)
    tail_iota = jax.lax.broadcasted_iota(jnp.int32, (1, TW), 1)
    corr = jnp.float32(0.0)
    for r in range(RB):
        t_r = tgt_sref[i * RB + r]
        start = jnp.minimum((t_r // LANE) * LANE, SMAXD)
        xg = buf[slot, pl.ds(r, 1), pl.ds(start, WIN)]
        val = jnp.sum(jnp.where(lane_iota == (t_r - start), xg, 0.0))
        xt = buf[slot, pl.ds(r, 1), TSTART:N_CLASSES]
        val = val + jnp.sum(jnp.where(tail_iota == (t_r - TSTART), xt, 0.0))
        corr = corr + jnp.where(t_r == IGNORE, 0.0, val)

    out_ref[0, 0] += -(fill_part + DELTA * corr)

    for q in range(CPB):
        idx = i * CPB + q

        @pl.when(idx < SC_ROWS)
        def _(idx=idx):
            _corr_dma(logit_hbm, corrbuf, csem, tgt_sref, idx).start()

    @pl.when(i == GRID - 1)
    def _():
        for idx in range(SC_ROWS):
            _corr_dma(logit_hbm, corrbuf, csem, tgt_sref, idx).wait()
        g_iota = jax.lax.broadcasted_iota(jnp.int32, (1, LANE), 1)
        corr_sc = jnp.float32(0.0)
        for idx in range(SC_ROWS):
            t_r = tgt_sref[TC_ROWS + idx]
            tcol = jnp.minimum((t_r // LANE) * LANE, SC_COLS - LANE)
            xrow = corrbuf[idx, pl.ds(idx % 8, 1), :]
            val = jnp.sum(jnp.where(g_iota == (t_r - tcol), xrow, 0.0))
            corr_sc = corr_sc + jnp.where(t_r == IGNORE, 0.0, val)
        xs = strip_ref[...]
        tsv = tsc_ref[...]
        col_s = jax.lax.broadcasted_iota(jnp.int32, xs.shape, 1) + SC_COLS
        fill_s = jnp.where(tsv == IGNORE, 0.0, FILL)
        fill_p = jnp.sum(fill_s * jnp.sum(xs, axis=1, keepdims=True))
        corr_s = jnp.sum(jnp.where(col_s == tsv, xs, 0.0))
        out_ref[0, 0] += -(fill_p + DELTA * (corr_s + corr_sc))

    @pl.when(i + NBUF < GRID)
    def _():
        _copy_in(logit_hbm, buf, sems, i + NBUF, slot)


def _combine_body(s1_ref, scpart_ref, tslot_ref, out_ref):
    rs = jnp.sum(scpart_ref[...], axis=1, keepdims=True)
    tsv = tslot_ref[...]
    fill_s = jnp.where(tsv == IGNORE, 0.0, FILL)
    out_ref[0, 0] = s1_ref[0, 0] - jnp.sum(fill_s * rs)


def kernel(logit, target):
    t1 = target.astype(jnp.int32)
    sc_part = _get_sc_slab()(logit)
    t2 = t1.reshape(N_ROWS, 1)
    s1 = pl.pallas_call(
        _loss_body,
        grid_spec=pltpu.PrefetchScalarGridSpec(
            num_scalar_prefetch=1,
            grid=(GRID,),
            in_specs=[
                pl.BlockSpec(memory_space=pltpu.HBM),
                pl.BlockSpec((RB, 1), lambda i, t_sref: (i, 0)),
                pl.BlockSpec((SC_ROWS, STRIP), lambda i, t_sref: (0, 0)),
                pl.BlockSpec((SC_ROWS, 1), lambda i, t_sref: (0, 0)),
            ],
            out_specs=pl.BlockSpec(memory_space=pltpu.SMEM),
            scratch_shapes=[
                pltpu.VMEM((NBUF, RB, N_CLASSES), jnp.float32),
                pltpu.SemaphoreType.DMA((NBUF,)),
                pltpu.VMEM((SC_ROWS, 8, LANE), jnp.float32),
                pltpu.SemaphoreType.DMA,
            ],
        ),
        out_shape=jax.ShapeDtypeStruct((1, 1), jnp.float32),
    )(t1, logit, t2, logit[TC_ROWS:, SC_COLS:], t2[TC_ROWS:])
    slot_rows = jnp.asarray(
        [(w // 2) * RPW + k for w in range(_NW) for k in range(RPW)],
        dtype=jnp.int32,
    )
    t_slot = t1[TC_ROWS:][slot_rows].reshape(NSLOT, 1)
    res = pl.pallas_call(
        _combine_body,
        in_specs=[
            pl.BlockSpec(memory_space=pltpu.SMEM),
            pl.BlockSpec((NSLOT, 16), lambda: (0, 0)),
            pl.BlockSpec((NSLOT, 1), lambda: (0, 0)),
        ],
        out_specs=pl.BlockSpec(memory_space=pltpu.SMEM),
        out_shape=jax.ShapeDtypeStruct((1, 1), jnp.float32),
    )(s1, sc_part.reshape(NSLOT, 16), t_slot)
    return res[0, 0]


# speedup vs baseline: 1.0590x; 1.0590x over previous
import jax
import jax.numpy as jnp
from jax.experimental import pallas as pl
from jax.experimental.pallas import tpu as pltpu

N_ROWS = 1024
N_CLASSES = 100000
IGNORE = 0
SMOOTH = 0.1
FILL = SMOOTH / (N_CLASSES - 1)
CONF = 1.0 - SMOOTH
DELTA = CONF - FILL

RB = 16
GRID = N_ROWS // RB
NBUF = 4
LANE = 128
NFULL = N_CLASSES // LANE
REM = N_CLASSES - NFULL * LANE
WIN = 512
TSTART = 99584
TW = N_CLASSES - TSTART
SMAXD = TSTART - WIN


def _copy_in(logit_hbm, buf, sems, blk, slot):
    pltpu.make_async_copy(
        logit_hbm.at[pl.ds(blk * RB, RB), :],
        buf.at[slot],
        sems.at[slot],
    ).start()


def _loss_body(tgt_sref, logit_hbm, tgt_ref, out_ref, buf, sems):
    i = pl.program_id(0)
    slot = jax.lax.rem(i, NBUF)

    @pl.when(i == 0)
    def _():
        out_ref[0, 0] = 0.0
        for b in range(NBUF):
            _copy_in(logit_hbm, buf, sems, jnp.int32(b), jnp.int32(b))

    pltpu.make_async_copy(
        logit_hbm.at[pl.ds(i * RB, RB), :],
        buf.at[slot],
        sems.at[slot],
    ).wait()

    x = buf[slot]

    rs_row = jnp.sum(x, axis=1, keepdims=True)
    t = tgt_ref[...]
    fill_row = jnp.where(t == IGNORE, 0.0, FILL)
    fill_part = jnp.sum(fill_row * rs_row)

    lane_iota = jax.lax.broadcasted_iota(jnp.int32, (1, WIN), 1)
    tail_iota = jax.lax.broadcasted_iota(jnp.int32, (1, TW), 1)
    corr = jnp.float32(0.0)
    for r in range(RB):
        t_r = tgt_sref[i * RB + r]
        start = jnp.minimum((t_r // LANE) * LANE, SMAXD)
        xg = buf[slot, pl.ds(r, 1), pl.ds(start, WIN)]
        val = jnp.sum(jnp.where(lane_iota == (t_r - start), xg, 0.0))
        xt = buf[slot, pl.ds(r, 1), TSTART:N_CLASSES]
        val = val + jnp.sum(jnp.where(tail_iota == (t_r - TSTART), xt, 0.0))
        corr = corr + jnp.where(t_r == IGNORE, 0.0, val)

    out_ref[0, 0] += -(fill_part + DELTA * corr)

    @pl.when(i + NBUF < GRID)
    def _():
        _copy_in(logit_hbm, buf, sems, i + NBUF, slot)


def kernel(logit, target):
    t1 = target.astype(jnp.int32)
    res = pl.pallas_call(
        _loss_body,
        grid_spec=pltpu.PrefetchScalarGridSpec(
            num_scalar_prefetch=1,
            grid=(GRID,),
            in_specs=[
                pl.BlockSpec(memory_space=pltpu.HBM),
                pl.BlockSpec((RB, 1), lambda i, t_sref: (i, 0)),
            ],
            out_specs=pl.BlockSpec(memory_space=pltpu.SMEM),
            scratch_shapes=[
                pltpu.VMEM((NBUF, RB, N_CLASSES), jnp.float32),
                pltpu.SemaphoreType.DMA((NBUF,)),
            ],
        ),
        out_shape=jax.ShapeDtypeStruct((1, 1), jnp.float32),
    )(t1, logit, t1.reshape(N_ROWS, 1))
    return res[0, 0]
